# Initial kernel scaffold; baseline (speedup 1.0000x reference)
import jax
import jax.numpy as jnp
from jax.experimental import pallas as pl


def kernel(h, edge_index, e, W_emb, b_emb, W1, b1, g1, bt1, W2, b2, g_app, b_app, g_gin, b_gin, eps):
    raise NotImplementedError("write your pallas kernel here")


# speedup vs baseline: 3.7536x; 3.7536x over previous
import jax
import jax.numpy as jnp
from jax import lax
from jax.experimental import pallas as pl
from jax.experimental.pallas import tpu as pltpu
from jax.experimental.pallas import tpu_sc as plsc

N = 10000
E = 320000
D = 128
L = 4
BN_EPS = 1e-5

NW = 32
CH = 128
EPW = E // NW
NCH = -(-EPW // CH)
EPAD = NW * NCH * CH
ROWS_ACC = 10112
RPT = ROWS_ACC // 16


def _sc_segment_sum_body(h_hbm, src_hbm, dst_hbm, zeros_hbm, out_hbm,
                         src_v, dst_v, rows_v, acc, sem):
    cid = lax.axis_index("c")
    sid = lax.axis_index("s")
    wid = sid * 2 + cid

    pltpu.sync_copy(src_hbm.at[wid], src_v)
    pltpu.sync_copy(dst_hbm.at[wid], dst_v)
    pltpu.sync_copy(zeros_hbm, acc.at[pl.ds(sid * RPT, RPT)])
    plsc.subcore_barrier()

    def chunk(c, carry):
        pltpu.async_copy(h_hbm.at[src_v.at[c]], rows_v, sem).wait()
        pltpu.sync_copy(rows_v, acc.at[dst_v.at[c]], add=True)
        return carry

    lax.fori_loop(0, NCH, chunk, 0)
    plsc.subcore_barrier()

    pltpu.sync_copy(acc.at[pl.ds(sid * RPT, RPT)],
                    out_hbm.at[cid, pl.ds(sid * RPT, RPT)])


_sc_segment_sum = pl.kernel(
    _sc_segment_sum_body,
    out_type=jax.ShapeDtypeStruct((2, ROWS_ACC, D), jnp.float32),
    mesh=plsc.VectorSubcoreMesh(core_axis_name="c", subcore_axis_name="s"),
    scratch_types=[
        pltpu.VMEM((NCH, CH), jnp.int32),
        pltpu.VMEM((NCH, CH), jnp.int32),
        pltpu.VMEM((CH, D), jnp.float32),
        pltpu.VMEM_SHARED((ROWS_ACC, D), jnp.float32),
        pltpu.SemaphoreType.DMA,
    ],
)


def _bn(x, gamma, beta):
    mean = jnp.mean(x, axis=0, keepdims=True)
    var = jnp.mean((x - mean) ** 2, axis=0, keepdims=True)
    return (x - mean) * lax.rsqrt(var + BN_EPS) * gamma + beta


def _embed_body(h_ref, w_ref, b_ref, out_ref):
    out_ref[...] = (
        jnp.dot(h_ref[...], w_ref[...], preferred_element_type=jnp.float32)
        + b_ref[...]
    )


_embed = pl.pallas_call(
    _embed_body,
    out_shape=jax.ShapeDtypeStruct((N, D), jnp.float32),
    compiler_params=pltpu.CompilerParams(vmem_limit_bytes=100 * 1024 * 1024),
)


def _layer_body(h_ref, n0_ref, n1_ref, w1_ref, b1_ref, g1_ref, bt1_ref,
                w2_ref, b2_ref, ga_ref, ba_ref, gg_ref, bg_ref, eps_ref,
                out_ref):
    h = h_ref[...]
    x = (1.0 + eps_ref[0, 0]) * h + (n0_ref[...] + n1_ref[...])
    x = jnp.dot(x, w1_ref[...], preferred_element_type=jnp.float32) + b1_ref[...]
    x = jnp.maximum(_bn(x, g1_ref[...], bt1_ref[...]), 0.0)
    x = jnp.dot(x, w2_ref[...], preferred_element_type=jnp.float32) + b2_ref[...]
    x = jnp.maximum(_bn(x, ga_ref[...], ba_ref[...]), 0.0)
    x = jnp.maximum(_bn(x, gg_ref[...], bg_ref[...]), 0.0)
    out_ref[...] = h + x


_layer = pl.pallas_call(
    _layer_body,
    out_shape=jax.ShapeDtypeStruct((N, D), jnp.float32),
    compiler_params=pltpu.CompilerParams(vmem_limit_bytes=100 * 1024 * 1024),
)


def kernel(h, edge_index, e, W_emb, b_emb, W1, b1, g1, bt1, W2, b2,
           g_app, b_app, g_gin, b_gin, eps):
    src = edge_index[0]
    dst = edge_index[1]
    pad = EPAD - E
    srcp = jnp.concatenate([src, jnp.zeros((pad,), jnp.int32)]).reshape(NW, NCH, CH)
    dstp = jnp.concatenate([dst, jnp.full((pad,), N, jnp.int32)]).reshape(NW, NCH, CH)
    zeros = jnp.zeros((RPT, D), jnp.float32)

    h = _embed(h, W_emb, b_emb.reshape(1, D))
    for l in range(L):
        parts = _sc_segment_sum(h, srcp, dstp, zeros)
        h = _layer(
            h, parts[0, :N], parts[1, :N],
            W1[l], b1[l].reshape(1, D), g1[l].reshape(1, D),
            bt1[l].reshape(1, D),
            W2[l], b2[l].reshape(1, D), g_app[l].reshape(1, D),
            b_app[l].reshape(1, D), g_gin[l].reshape(1, D),
            b_gin[l].reshape(1, D), eps[l].reshape(1, 1),
        )
    return h
